# Initial kernel scaffold; baseline (speedup 1.0000x reference)
import jax
import jax.numpy as jnp
from jax.experimental import pallas as pl


def kernel(x, edge_index, W1_rel, b1_rel, W1_root, W2_rel, b2_rel, W2_root):
    raise NotImplementedError("write your pallas kernel here")


# speedup vs baseline: 5.3186x; 5.3186x over previous
import functools

import jax
import jax.numpy as jnp
from jax import lax
from jax.experimental import pallas as pl
from jax.experimental.pallas import tpu as pltpu
from jax.experimental.pallas import tpu_sc as plsc

N = 10000
E = 320000
D_IN = 128
D_H = 64
D_OUT = 128

N_PAD = 10240
BLK = 512
GRID = N_PAD // BLK
NC, NS = 2, 16
NW = NC * NS
CHUNK = 128
CPW = -(-E // (NW * CHUNK))
E_PAD = NW * CPW * CHUNK
STRIPE = N_PAD // NS
W1A = D_H + 16


def _tc1_body(x_ref, wrel_ref, wroot_ref, yaug_ref, r1_ref):
    xb = x_ref[...]
    y = lax.dot_general(xb, wrel_ref[...], (((1,), (1,)), ((), ())),
                        preferred_element_type=jnp.float32)
    ones_col = (lax.broadcasted_iota(jnp.int32, (BLK, 16), 1) == 0)
    yaug_ref[...] = jnp.concatenate([y, ones_col.astype(jnp.float32)], axis=1)
    r1_ref[...] = lax.dot_general(xb, wroot_ref[...], (((1,), (1,)), ((), ())),
                                  preferred_element_type=jnp.float32)


def _tc1(x_pad, w1_rel, w1_root):
    return pl.pallas_call(
        _tc1_body,
        grid=(GRID,),
        in_specs=[
            pl.BlockSpec((BLK, D_IN), lambda i: (i, 0)),
            pl.BlockSpec((D_H, D_IN), lambda i: (0, 0)),
            pl.BlockSpec((D_H, D_IN), lambda i: (0, 0)),
        ],
        out_specs=[
            pl.BlockSpec((BLK, W1A), lambda i: (i, 0)),
            pl.BlockSpec((BLK, D_H), lambda i: (i, 0)),
        ],
        out_shape=[
            jax.ShapeDtypeStruct((N_PAD, W1A), jnp.float32),
            jax.ShapeDtypeStruct((N_PAD, D_H), jnp.float32),
        ],
    )(x_pad, w1_rel, w1_root)


def _tc2_body(acc_ref, r1_ref, b1_ref, ms_ref, h_ref):
    a0 = acc_ref[0]
    a1 = acc_ref[1]
    seg = a0[:, :D_H] + a1[:, :D_H]
    deg = a0[:, D_H:D_H + 1] + a1[:, D_H:D_H + 1]
    degc = jnp.maximum(deg, 1.0)
    h = seg / degc + b1_ref[...] + r1_ref[...]
    h_ref[...] = jnp.maximum(h, 0.0) * ms_ref[...]


def _tc2(acc1, r1, b1r, ms_pad):
    return pl.pallas_call(
        _tc2_body,
        grid=(GRID,),
        in_specs=[
            pl.BlockSpec((2, BLK, W1A), lambda i: (0, i, 0)),
            pl.BlockSpec((BLK, D_H), lambda i: (i, 0)),
            pl.BlockSpec((1, D_H), lambda i: (0, 0)),
            pl.BlockSpec((BLK, D_H), lambda i: (i, 0)),
        ],
        out_specs=pl.BlockSpec((BLK, D_H), lambda i: (i, 0)),
        out_shape=jax.ShapeDtypeStruct((N_PAD, D_H), jnp.float32),
    )(acc1, r1, b1r, ms_pad)


def _tc3_body(acc_ref, h_ref, wrel_ref, b2_ref, wroot_ref, out_ref):
    seg2 = acc_ref[0] + acc_ref[1]
    out = lax.dot_general(seg2, wrel_ref[...], (((1,), (1,)), ((), ())),
                          preferred_element_type=jnp.float32)
    out += lax.dot_general(h_ref[...], wroot_ref[...], (((1,), (1,)), ((), ())),
                           preferred_element_type=jnp.float32)
    out_ref[...] = out + b2_ref[...]


def _tc3(acc2, h, w2_rel, b2r, w2_root):
    return pl.pallas_call(
        _tc3_body,
        grid=(GRID,),
        in_specs=[
            pl.BlockSpec((2, BLK, D_H), lambda i: (0, i, 0)),
            pl.BlockSpec((BLK, D_H), lambda i: (i, 0)),
            pl.BlockSpec((D_OUT, D_H), lambda i: (0, 0)),
            pl.BlockSpec((1, D_OUT), lambda i: (0, 0)),
            pl.BlockSpec((D_OUT, D_H), lambda i: (0, 0)),
        ],
        out_specs=pl.BlockSpec((BLK, D_OUT), lambda i: (i, 0)),
        out_shape=jax.ShapeDtypeStruct((N_PAD, D_OUT), jnp.float32),
    )(acc2, h, w2_rel, b2r, w2_root)


def _make_sc_seg_sum(width):
    mesh = plsc.VectorSubcoreMesh(core_axis_name="c", subcore_axis_name="s")

    @functools.partial(
        pl.kernel,
        mesh=mesh,
        compiler_params=pltpu.CompilerParams(use_tc_tiling_on_sc=False),
        out_type=jax.ShapeDtypeStruct((NC, N_PAD, width), jnp.float32),
        scratch_types=[
            pltpu.VMEM((CHUNK,), jnp.int32),
            pltpu.VMEM((CHUNK,), jnp.int32),
            pltpu.VMEM((CHUNK, width), jnp.float32),
            pltpu.VMEM_SHARED((N_PAD, width), jnp.float32),
            pltpu.SemaphoreType.DMA,
        ],
    )
    def seg_sum(table_hbm, src_hbm, dst_hbm, out_hbm,
                src_v, dst_v, rows_v, acc_sh, sem):
        c = lax.axis_index("c")
        s = lax.axis_index("s")
        wid = s * NC + c

        def _zero_row(i, carry):
            for jj in range(width // 16):
                rows_v[i, pl.ds(jj * 16, 16)] = jnp.zeros((16,), jnp.float32)
            return carry

        lax.fori_loop(0, CHUNK, _zero_row, 0)
        for k in range(STRIPE // CHUNK):
            pltpu.sync_copy(rows_v, acc_sh.at[pl.ds(s * STRIPE + k * CHUNK, CHUNK)])
        plsc.subcore_barrier()

        def _step(j, carry):
            g = wid * CPW + j
            pltpu.sync_copy(src_hbm.at[g], src_v)
            pltpu.sync_copy(dst_hbm.at[g], dst_v)
            pltpu.async_copy(table_hbm.at[src_v], rows_v, sem).wait()
            pltpu.sync_copy(rows_v, acc_sh.at[dst_v], add=True)
            return carry

        lax.fori_loop(0, CPW, _step, 0)
        plsc.subcore_barrier()

        pltpu.sync_copy(acc_sh.at[pl.ds(s * STRIPE, STRIPE)],
                        out_hbm.at[c, pl.ds(s * STRIPE, STRIPE)])

    return seg_sum


_sc_seg1 = _make_sc_seg_sum(W1A)
_sc_seg2 = _make_sc_seg_sum(D_H)


def kernel(x, edge_index, W1_rel, b1_rel, W1_root, W2_rel, b2_rel, W2_root):
    x_pad = jnp.pad(x, ((0, N_PAD - N), (0, 0)))
    src = jnp.concatenate(
        [edge_index[0], jnp.full((E_PAD - E,), N, jnp.int32)]).reshape(-1, CHUNK)
    dst = jnp.concatenate(
        [edge_index[1], jnp.full((E_PAD - E,), N, jnp.int32)]).reshape(-1, CHUNK)

    ms = 2.0 * jax.random.bernoulli(
        jax.random.key(42), 0.5, (N, D_H)).astype(jnp.float32)
    ms_pad = jnp.pad(ms, ((0, N_PAD - N), (0, 0)))

    yaug, r1 = _tc1(x_pad, W1_rel, W1_root)
    acc1 = _sc_seg1(yaug, src, dst)
    h = _tc2(acc1, r1, b1_rel.reshape(1, D_H), ms_pad)
    acc2 = _sc_seg2(h, src, dst)
    out = _tc3(acc2, h, W2_rel, b2_rel.reshape(1, D_OUT), W2_root)
    return out[:N]


# speedup vs baseline: 12.0206x; 2.2601x over previous
import functools

import jax
import jax.numpy as jnp
from jax import lax
from jax.experimental import pallas as pl
from jax.experimental.pallas import tpu as pltpu
from jax.experimental.pallas import tpu_sc as plsc

N = 10000
E = 320000
D_IN = 128
D_H = 64
D_OUT = 128

N_PAD = 10240
BLK = 512
GRID = N_PAD // BLK
NC, NS = 2, 16
NW = NC * NS
CHUNK = 128
CPW = 80
E_PAD = NW * CPW * CHUNK
STRIPE = N_PAD // NS
W1A = D_H + 16


def _tc1_body(x_ref, wrel_ref, wroot_ref, yaug_ref, r1_ref):
    xb = x_ref[...]
    y = lax.dot_general(xb, wrel_ref[...], (((1,), (1,)), ((), ())),
                        preferred_element_type=jnp.float32)
    ones_col = (lax.broadcasted_iota(jnp.int32, (BLK, 16), 1) == 0)
    yaug_ref[...] = jnp.concatenate([y, ones_col.astype(jnp.float32)], axis=1)
    r1_ref[...] = lax.dot_general(xb, wroot_ref[...], (((1,), (1,)), ((), ())),
                                  preferred_element_type=jnp.float32)


def _tc1(x_pad, w1_rel, w1_root):
    return pl.pallas_call(
        _tc1_body,
        grid=(GRID,),
        in_specs=[
            pl.BlockSpec((BLK, D_IN), lambda i: (i, 0)),
            pl.BlockSpec((D_H, D_IN), lambda i: (0, 0)),
            pl.BlockSpec((D_H, D_IN), lambda i: (0, 0)),
        ],
        out_specs=[
            pl.BlockSpec((BLK, W1A), lambda i: (i, 0)),
            pl.BlockSpec((BLK, D_H), lambda i: (i, 0)),
        ],
        out_shape=[
            jax.ShapeDtypeStruct((N_PAD, W1A), jnp.float32),
            jax.ShapeDtypeStruct((N_PAD, D_H), jnp.float32),
        ],
    )(x_pad, w1_rel, w1_root)


def _tc2_body(acc_ref, r1_ref, b1_ref, ms_ref, h_ref):
    a0 = acc_ref[0]
    a1 = acc_ref[1]
    seg = a0[:, :D_H] + a1[:, :D_H]
    deg = a0[:, D_H:D_H + 1] + a1[:, D_H:D_H + 1]
    degc = jnp.maximum(deg, 1.0)
    h = seg / degc + b1_ref[...] + r1_ref[...]
    h_ref[...] = jnp.maximum(h, 0.0) * ms_ref[...]


def _tc2(acc1, r1, b1r, ms_pad):
    return pl.pallas_call(
        _tc2_body,
        grid=(GRID,),
        in_specs=[
            pl.BlockSpec((2, BLK, W1A), lambda i: (0, i, 0)),
            pl.BlockSpec((BLK, D_H), lambda i: (i, 0)),
            pl.BlockSpec((1, D_H), lambda i: (0, 0)),
            pl.BlockSpec((BLK, D_H), lambda i: (i, 0)),
        ],
        out_specs=pl.BlockSpec((BLK, D_H), lambda i: (i, 0)),
        out_shape=jax.ShapeDtypeStruct((N_PAD, D_H), jnp.float32),
    )(acc1, r1, b1r, ms_pad)


def _tc3_body(acc_ref, h_ref, wrel_ref, b2_ref, wroot_ref, out_ref):
    seg2 = acc_ref[0] + acc_ref[1]
    out = lax.dot_general(seg2, wrel_ref[...], (((1,), (1,)), ((), ())),
                          preferred_element_type=jnp.float32)
    out += lax.dot_general(h_ref[...], wroot_ref[...], (((1,), (1,)), ((), ())),
                           preferred_element_type=jnp.float32)
    out_ref[...] = out + b2_ref[...]


def _tc3(acc2, h, w2_rel, b2r, w2_root):
    return pl.pallas_call(
        _tc3_body,
        grid=(GRID,),
        in_specs=[
            pl.BlockSpec((2, BLK, D_H), lambda i: (0, i, 0)),
            pl.BlockSpec((BLK, D_H), lambda i: (i, 0)),
            pl.BlockSpec((D_OUT, D_H), lambda i: (0, 0)),
            pl.BlockSpec((1, D_OUT), lambda i: (0, 0)),
            pl.BlockSpec((D_OUT, D_H), lambda i: (0, 0)),
        ],
        out_specs=pl.BlockSpec((BLK, D_OUT), lambda i: (i, 0)),
        out_shape=jax.ShapeDtypeStruct((N_PAD, D_OUT), jnp.float32),
    )(acc2, h, w2_rel, b2r, w2_root)


def _make_sc_seg_sum(width):
    mesh = plsc.VectorSubcoreMesh(core_axis_name="c", subcore_axis_name="s")

    @functools.partial(
        pl.kernel,
        mesh=mesh,
        compiler_params=pltpu.CompilerParams(use_tc_tiling_on_sc=False),
        out_type=jax.ShapeDtypeStruct((NC, N_PAD, width), jnp.float32),
        scratch_types=[
            pltpu.VMEM((CPW, CHUNK), jnp.int32),
            pltpu.VMEM((CPW, CHUNK), jnp.int32),
            pltpu.VMEM((CHUNK, width), jnp.float32),
            pltpu.VMEM((CHUNK, width), jnp.float32),
            pltpu.VMEM_SHARED((N_PAD, width), jnp.float32),
            pltpu.SemaphoreType.DMA,
            pltpu.SemaphoreType.DMA,
        ],
    )
    def seg_sum(table_hbm, src_hbm, dst_hbm, out_hbm,
                src_all, dst_all, rows0, rows1, acc_sh, sem0, sem1):
        c = lax.axis_index("c")
        s = lax.axis_index("s")
        wid = s * NC + c

        def _zero_row(i, carry):
            for jj in range(width // 16):
                rows0[i, pl.ds(jj * 16, 16)] = jnp.zeros((16,), jnp.float32)
            return carry

        lax.fori_loop(0, CHUNK, _zero_row, 0)
        for k in range(STRIPE // CHUNK):
            pltpu.sync_copy(rows0, acc_sh.at[pl.ds(s * STRIPE + k * CHUNK, CHUNK)])

        pltpu.sync_copy(src_hbm.at[pl.ds(wid * CPW, CPW)], src_all)
        pltpu.sync_copy(dst_hbm.at[pl.ds(wid * CPW, CPW)], dst_all)
        plsc.subcore_barrier()

        pltpu.async_copy(table_hbm.at[src_all.at[0]], rows0, sem0)

        def _pair(i, carry):
            ja = 2 * i
            jb = 2 * i + 1
            jc = jnp.minimum(jb + 1, CPW - 1)
            pltpu.make_async_copy(table_hbm.at[src_all.at[ja]], rows0, sem0).wait()
            pltpu.async_copy(table_hbm.at[src_all.at[jb]], rows1, sem1)
            pltpu.sync_copy(rows0, acc_sh.at[dst_all.at[ja]], add=True)
            pltpu.make_async_copy(table_hbm.at[src_all.at[jb]], rows1, sem1).wait()
            pltpu.async_copy(table_hbm.at[src_all.at[jc]], rows0, sem0)
            pltpu.sync_copy(rows1, acc_sh.at[dst_all.at[jb]], add=True)
            return carry

        lax.fori_loop(0, CPW // 2, _pair, 0)
        pltpu.make_async_copy(table_hbm.at[src_all.at[CPW - 1]], rows0, sem0).wait()
        plsc.subcore_barrier()

        pltpu.sync_copy(acc_sh.at[pl.ds(s * STRIPE, STRIPE)],
                        out_hbm.at[c, pl.ds(s * STRIPE, STRIPE)])

    return seg_sum


_sc_seg1 = _make_sc_seg_sum(W1A)
_sc_seg2 = _make_sc_seg_sum(D_H)


def kernel(x, edge_index, W1_rel, b1_rel, W1_root, W2_rel, b2_rel, W2_root):
    x_pad = jnp.pad(x, ((0, N_PAD - N), (0, 0)))
    pad_idx = N + jnp.arange(E_PAD - E, dtype=jnp.int32) % (N_PAD - N)
    src = jnp.concatenate([edge_index[0], pad_idx]).reshape(-1, CHUNK)
    dst = jnp.concatenate([edge_index[1], pad_idx]).reshape(-1, CHUNK)

    ms = 2.0 * jax.random.bernoulli(
        jax.random.key(42), 0.5, (N, D_H)).astype(jnp.float32)
    ms_pad = jnp.pad(ms, ((0, N_PAD - N), (0, 0)))

    yaug, r1 = _tc1(x_pad, W1_rel, W1_root)
    acc1 = _sc_seg1(yaug, src, dst)
    h = _tc2(acc1, r1, b1_rel.reshape(1, D_H), ms_pad)
    acc2 = _sc_seg2(h, src, dst)
    out = _tc3(acc2, h, W2_rel, b2_rel.reshape(1, D_OUT), W2_root)
    return out[:N]


# speedup vs baseline: 16.0500x; 1.3352x over previous
import functools

import jax
import jax.numpy as jnp
from jax import lax
from jax.experimental import pallas as pl
from jax.experimental.pallas import tpu as pltpu
from jax.experimental.pallas import tpu_sc as plsc

N = 10000
E = 320000
D_IN = 128
D_H = 64
D_OUT = 128

N_PAD = 10240
BLK = 512
GRID = N_PAD // BLK
NC, NS = 2, 16
NW = NC * NS
CHUNK = 128
CPW = 80
E_PAD = NW * CPW * CHUNK
STRIPE = N_PAD // NS
W1A = D_H + 16


def _tc1_body(x_ref, wrel_ref, wroot_ref, yaug_ref, r1_ref):
    xb = x_ref[...]
    y = lax.dot_general(xb, wrel_ref[...], (((1,), (1,)), ((), ())),
                        preferred_element_type=jnp.float32)
    ones_col = (lax.broadcasted_iota(jnp.int32, (BLK, 16), 1) == 0)
    yaug_ref[...] = jnp.concatenate([y, ones_col.astype(jnp.float32)], axis=1)
    r1_ref[...] = lax.dot_general(xb, wroot_ref[...], (((1,), (1,)), ((), ())),
                                  preferred_element_type=jnp.float32)


def _tc1(x_pad, w1_rel, w1_root):
    return pl.pallas_call(
        _tc1_body,
        grid=(GRID,),
        in_specs=[
            pl.BlockSpec((BLK, D_IN), lambda i: (i, 0)),
            pl.BlockSpec((D_H, D_IN), lambda i: (0, 0)),
            pl.BlockSpec((D_H, D_IN), lambda i: (0, 0)),
        ],
        out_specs=[
            pl.BlockSpec((BLK, W1A), lambda i: (i, 0)),
            pl.BlockSpec((BLK, D_H), lambda i: (i, 0)),
        ],
        out_shape=[
            jax.ShapeDtypeStruct((N_PAD, W1A), jnp.float32),
            jax.ShapeDtypeStruct((N_PAD, D_H), jnp.float32),
        ],
    )(x_pad, w1_rel, w1_root)


def _tc2_body(acc_ref, r1_ref, b1_ref, ms_ref, h_ref):
    a0 = acc_ref[0]
    a1 = acc_ref[1]
    seg = a0[:, :D_H] + a1[:, :D_H]
    deg = a0[:, D_H:D_H + 1] + a1[:, D_H:D_H + 1]
    degc = jnp.maximum(deg, 1.0)
    h = seg / degc + b1_ref[...] + r1_ref[...]
    h_ref[...] = jnp.maximum(h, 0.0) * ms_ref[...]


def _tc2(acc1, r1, b1r, ms_pad):
    return pl.pallas_call(
        _tc2_body,
        grid=(GRID,),
        in_specs=[
            pl.BlockSpec((2, BLK, W1A), lambda i: (0, i, 0)),
            pl.BlockSpec((BLK, D_H), lambda i: (i, 0)),
            pl.BlockSpec((1, D_H), lambda i: (0, 0)),
            pl.BlockSpec((BLK, D_H), lambda i: (i, 0)),
        ],
        out_specs=pl.BlockSpec((BLK, D_H), lambda i: (i, 0)),
        out_shape=jax.ShapeDtypeStruct((N_PAD, D_H), jnp.float32),
    )(acc1, r1, b1r, ms_pad)


def _tc3_body(acc_ref, h_ref, wrel_ref, b2_ref, wroot_ref, out_ref):
    seg2 = acc_ref[0] + acc_ref[1]
    out = lax.dot_general(seg2, wrel_ref[...], (((1,), (1,)), ((), ())),
                          preferred_element_type=jnp.float32)
    out += lax.dot_general(h_ref[...], wroot_ref[...], (((1,), (1,)), ((), ())),
                           preferred_element_type=jnp.float32)
    out_ref[...] = out + b2_ref[...]


def _tc3(acc2, h, w2_rel, b2r, w2_root):
    return pl.pallas_call(
        _tc3_body,
        grid=(GRID,),
        in_specs=[
            pl.BlockSpec((2, BLK, D_H), lambda i: (0, i, 0)),
            pl.BlockSpec((BLK, D_H), lambda i: (i, 0)),
            pl.BlockSpec((D_OUT, D_H), lambda i: (0, 0)),
            pl.BlockSpec((1, D_OUT), lambda i: (0, 0)),
            pl.BlockSpec((D_OUT, D_H), lambda i: (0, 0)),
        ],
        out_specs=pl.BlockSpec((BLK, D_OUT), lambda i: (i, 0)),
        out_shape=jax.ShapeDtypeStruct((N_PAD, D_OUT), jnp.float32),
    )(acc2, h, w2_rel, b2r, w2_root)


def _make_sc_seg_sum(width):
    mesh = plsc.VectorSubcoreMesh(core_axis_name="c", subcore_axis_name="s")

    @functools.partial(
        pl.kernel,
        mesh=mesh,
        compiler_params=pltpu.CompilerParams(use_tc_tiling_on_sc=False),
        out_type=jax.ShapeDtypeStruct((NC, N_PAD, width), jnp.float32),
        scratch_types=[
            pltpu.VMEM((CPW, CHUNK), jnp.int32),
            pltpu.VMEM((CPW, CHUNK), jnp.int32),
            [pltpu.VMEM((CHUNK, width), jnp.float32)] * 4,
            pltpu.VMEM_SHARED((N_PAD, width), jnp.float32),
            [pltpu.SemaphoreType.DMA] * 4,
        ],
    )
    def seg_sum(table_hbm, src_hbm, dst_hbm, out_hbm,
                src_all, dst_all, rows, acc_sh, gsem):
        c = lax.axis_index("c")
        s = lax.axis_index("s")
        wid = s * NC + c

        def _zero_row(i, carry):
            for jj in range(width // 16):
                rows[0][i, pl.ds(jj * 16, 16)] = jnp.zeros((16,), jnp.float32)
            return carry

        lax.fori_loop(0, CHUNK, _zero_row, 0)
        for k in range(STRIPE // CHUNK):
            pltpu.sync_copy(rows[0], acc_sh.at[pl.ds(s * STRIPE + k * CHUNK, CHUNK)])

        pltpu.sync_copy(src_hbm.at[pl.ds(wid * CPW, CPW)], src_all)
        pltpu.sync_copy(dst_hbm.at[pl.ds(wid * CPW, CPW)], dst_all)
        plsc.subcore_barrier()

        def _gather(j, b):
            pltpu.async_copy(table_hbm.at[src_all.at[j]], rows[b], gsem[b])

        def _gwait(j, b):
            pltpu.make_async_copy(
                table_hbm.at[src_all.at[j]], rows[b], gsem[b]).wait()

        def _scatter(j, b):
            pltpu.sync_copy(rows[b], acc_sh.at[dst_all.at[j]], add=True)

        def _step(j, b, fire_ahead):
            if fire_ahead:
                _gather(j + 3, (b + 3) % 4)
            _gwait(j, b)
            _scatter(j, b)

        _gather(0, 0)
        _gather(1, 1)
        _gather(2, 2)
        for b in range(4):
            _step(b, b, fire_ahead=True)

        def _quad(q, carry):
            for b in range(4):
                _step(4 * q + b, b, fire_ahead=True)
            return carry

        lax.fori_loop(1, CPW // 4 - 1, _quad, 0)

        for b in range(4):
            _step(CPW - 4 + b, b, fire_ahead=(b < 1))
        plsc.subcore_barrier()

        pltpu.sync_copy(acc_sh.at[pl.ds(s * STRIPE, STRIPE)],
                        out_hbm.at[c, pl.ds(s * STRIPE, STRIPE)])

    return seg_sum


_sc_seg1 = _make_sc_seg_sum(W1A)
_sc_seg2 = _make_sc_seg_sum(D_H)


def kernel(x, edge_index, W1_rel, b1_rel, W1_root, W2_rel, b2_rel, W2_root):
    x_pad = jnp.pad(x, ((0, N_PAD - N), (0, 0)))
    pad_idx = N + jnp.arange(E_PAD - E, dtype=jnp.int32) % (N_PAD - N)
    src = jnp.concatenate([edge_index[0], pad_idx]).reshape(-1, CHUNK)
    dst = jnp.concatenate([edge_index[1], pad_idx]).reshape(-1, CHUNK)

    ms = 2.0 * jax.random.bernoulli(
        jax.random.key(42), 0.5, (N, D_H)).astype(jnp.float32)
    ms_pad = jnp.pad(ms, ((0, N_PAD - N), (0, 0)))

    yaug, r1 = _tc1(x_pad, W1_rel, W1_root)
    acc1 = _sc_seg1(yaug, src, dst)
    h = _tc2(acc1, r1, b1_rel.reshape(1, D_H), ms_pad)
    acc2 = _sc_seg2(h, src, dst)
    out = _tc3(acc2, h, W2_rel, b2_rel.reshape(1, D_OUT), W2_root)
    return out[:N]


# speedup vs baseline: 17.7183x; 1.1039x over previous
import functools

import jax
import jax.numpy as jnp
from jax import lax
from jax.experimental import pallas as pl
from jax.experimental.pallas import tpu as pltpu
from jax.experimental.pallas import tpu_sc as plsc

N = 10000
E = 320000
D_IN = 128
D_H = 64
D_OUT = 128

N_PAD = 10240
BLK = 2048
GRID = N_PAD // BLK
NC, NS = 2, 16
NW = NC * NS
CHUNK = 128
CPW = 80
E_PAD = NW * CPW * CHUNK
STRIPE = N_PAD // NS
W1A = D_H + 16


def _tc1_body(x_ref, wrel_ref, wroot_ref, yaug_ref, r1_ref):
    xb = x_ref[...]
    y = lax.dot_general(xb, wrel_ref[...], (((1,), (1,)), ((), ())),
                        preferred_element_type=jnp.float32)
    ones_col = (lax.broadcasted_iota(jnp.int32, (BLK, 16), 1) == 0)
    yaug_ref[...] = jnp.concatenate([y, ones_col.astype(jnp.float32)], axis=1)
    r1_ref[...] = lax.dot_general(xb, wroot_ref[...], (((1,), (1,)), ((), ())),
                                  preferred_element_type=jnp.float32)


def _tc1(x_pad, w1_rel, w1_root):
    return pl.pallas_call(
        _tc1_body,
        grid=(GRID,),
        in_specs=[
            pl.BlockSpec((BLK, D_IN), lambda i: (i, 0)),
            pl.BlockSpec((D_H, D_IN), lambda i: (0, 0)),
            pl.BlockSpec((D_H, D_IN), lambda i: (0, 0)),
        ],
        out_specs=[
            pl.BlockSpec((BLK, W1A), lambda i: (i, 0)),
            pl.BlockSpec((BLK, D_H), lambda i: (i, 0)),
        ],
        out_shape=[
            jax.ShapeDtypeStruct((N_PAD, W1A), jnp.float32),
            jax.ShapeDtypeStruct((N_PAD, D_H), jnp.float32),
        ],
    )(x_pad, w1_rel, w1_root)


def _tc2_body(acc_ref, r1_ref, b1_ref, ms_ref, h_ref):
    a0 = acc_ref[0]
    a1 = acc_ref[1]
    seg = a0[:, :D_H] + a1[:, :D_H]
    deg = a0[:, D_H:D_H + 1] + a1[:, D_H:D_H + 1]
    degc = jnp.maximum(deg, 1.0)
    h = seg / degc + b1_ref[...] + r1_ref[...]
    h_ref[...] = jnp.maximum(h, 0.0) * ms_ref[...]


def _tc2(acc1, r1, b1r, ms_pad):
    return pl.pallas_call(
        _tc2_body,
        grid=(GRID,),
        in_specs=[
            pl.BlockSpec((2, BLK, W1A), lambda i: (0, i, 0)),
            pl.BlockSpec((BLK, D_H), lambda i: (i, 0)),
            pl.BlockSpec((1, D_H), lambda i: (0, 0)),
            pl.BlockSpec((BLK, D_H), lambda i: (i, 0)),
        ],
        out_specs=pl.BlockSpec((BLK, D_H), lambda i: (i, 0)),
        out_shape=jax.ShapeDtypeStruct((N_PAD, D_H), jnp.float32),
    )(acc1, r1, b1r, ms_pad)


def _tc3_body(acc_ref, h_ref, wrel_ref, b2_ref, wroot_ref, out_ref):
    seg2 = acc_ref[0] + acc_ref[1]
    out = lax.dot_general(seg2, wrel_ref[...], (((1,), (1,)), ((), ())),
                          preferred_element_type=jnp.float32)
    out += lax.dot_general(h_ref[...], wroot_ref[...], (((1,), (1,)), ((), ())),
                           preferred_element_type=jnp.float32)
    out_ref[...] = out + b2_ref[...]


def _tc3(acc2, h, w2_rel, b2r, w2_root):
    return pl.pallas_call(
        _tc3_body,
        grid=(GRID,),
        in_specs=[
            pl.BlockSpec((2, BLK, D_H), lambda i: (0, i, 0)),
            pl.BlockSpec((BLK, D_H), lambda i: (i, 0)),
            pl.BlockSpec((D_OUT, D_H), lambda i: (0, 0)),
            pl.BlockSpec((1, D_OUT), lambda i: (0, 0)),
            pl.BlockSpec((D_OUT, D_H), lambda i: (0, 0)),
        ],
        out_specs=pl.BlockSpec((BLK, D_OUT), lambda i: (i, 0)),
        out_shape=jax.ShapeDtypeStruct((N_PAD, D_OUT), jnp.float32),
    )(acc2, h, w2_rel, b2r, w2_root)


def _make_sc_seg_sum(width):
    mesh = plsc.VectorSubcoreMesh(core_axis_name="c", subcore_axis_name="s")

    @functools.partial(
        pl.kernel,
        mesh=mesh,
        compiler_params=pltpu.CompilerParams(use_tc_tiling_on_sc=False),
        out_type=jax.ShapeDtypeStruct((NC, N_PAD, width), jnp.float32),
        scratch_types=[
            pltpu.VMEM((CPW, CHUNK), jnp.int32),
            pltpu.VMEM((CPW, CHUNK), jnp.int32),
            [pltpu.VMEM((CHUNK, width), jnp.float32)] * 4,
            pltpu.VMEM_SHARED((N_PAD, width), jnp.float32),
            [pltpu.SemaphoreType.DMA] * 4,
        ],
    )
    def seg_sum(table_hbm, src_hbm, dst_hbm, out_hbm,
                src_all, dst_all, rows, acc_sh, gsem):
        c = lax.axis_index("c")
        s = lax.axis_index("s")
        wid = s * NC + c

        def _zero_row(i, carry):
            for jj in range(width // 16):
                rows[0][i, pl.ds(jj * 16, 16)] = jnp.zeros((16,), jnp.float32)
            return carry

        lax.fori_loop(0, CHUNK, _zero_row, 0)
        for k in range(STRIPE // CHUNK):
            pltpu.sync_copy(rows[0], acc_sh.at[pl.ds(s * STRIPE + k * CHUNK, CHUNK)])

        pltpu.sync_copy(src_hbm.at[pl.ds(wid * CPW, CPW)], src_all)
        pltpu.sync_copy(dst_hbm.at[pl.ds(wid * CPW, CPW)], dst_all)
        plsc.subcore_barrier()

        def _gather(j, b):
            pltpu.async_copy(table_hbm.at[src_all.at[j]], rows[b], gsem[b])

        def _gwait(j, b):
            pltpu.make_async_copy(
                table_hbm.at[src_all.at[j]], rows[b], gsem[b]).wait()

        def _scatter(j, b):
            pltpu.sync_copy(rows[b], acc_sh.at[dst_all.at[j]], add=True)

        def _step(j, b, fire_ahead):
            if fire_ahead:
                _gather(j + 3, (b + 3) % 4)
            _gwait(j, b)
            _scatter(j, b)

        _gather(0, 0)
        _gather(1, 1)
        _gather(2, 2)
        for b in range(4):
            _step(b, b, fire_ahead=True)

        def _quad(q, carry):
            for b in range(4):
                _step(4 * q + b, b, fire_ahead=True)
            return carry

        lax.fori_loop(1, CPW // 4 - 1, _quad, 0)

        for b in range(4):
            _step(CPW - 4 + b, b, fire_ahead=(b < 1))
        plsc.subcore_barrier()

        pltpu.sync_copy(acc_sh.at[pl.ds(s * STRIPE, STRIPE)],
                        out_hbm.at[c, pl.ds(s * STRIPE, STRIPE)])

    return seg_sum


_sc_seg1 = _make_sc_seg_sum(W1A)
_sc_seg2 = _make_sc_seg_sum(D_H)


def kernel(x, edge_index, W1_rel, b1_rel, W1_root, W2_rel, b2_rel, W2_root):
    x_pad = jnp.pad(x, ((0, N_PAD - N), (0, 0)))
    pad_idx = N + jnp.arange(E_PAD - E, dtype=jnp.int32) % (N_PAD - N)
    src = jnp.concatenate([edge_index[0], pad_idx]).reshape(-1, CHUNK)
    dst = jnp.concatenate([edge_index[1], pad_idx]).reshape(-1, CHUNK)

    ms = 2.0 * jax.random.bernoulli(
        jax.random.key(42), 0.5, (N, D_H)).astype(jnp.float32)
    ms_pad = jnp.pad(ms, ((0, N_PAD - N), (0, 0)))

    yaug, r1 = _tc1(x_pad, W1_rel, W1_root)
    acc1 = _sc_seg1(yaug, src, dst)
    h = _tc2(acc1, r1, b1_rel.reshape(1, D_H), ms_pad)
    acc2 = _sc_seg2(h, src, dst)
    out = _tc3(acc2, h, W2_rel, b2_rel.reshape(1, D_OUT), W2_root)
    return out[:N]


# speedup vs baseline: 19.6318x; 1.1080x over previous
import functools

import jax
import jax.numpy as jnp
from jax import lax
from jax.experimental import pallas as pl
from jax.experimental.pallas import tpu as pltpu
from jax.experimental.pallas import tpu_sc as plsc

N = 10000
E = 320000
D_IN = 128
D_H = 64
D_OUT = 128

N_PAD = 10240
BLK = 2048
GRID = N_PAD // BLK
NC, NS = 2, 16
NW = NC * NS
CHUNK = 128
CPW = 80
E_PAD = NW * CPW * CHUNK
STRIPE = N_PAD // NS
def _tc1_body(x_ref, wrel_ref, wroot_ref, y1_ref, r1_ref):
    xb = x_ref[...]
    y1_ref[...] = lax.dot_general(xb, wrel_ref[...], (((1,), (1,)), ((), ())),
                                  preferred_element_type=jnp.float32)
    r1_ref[...] = lax.dot_general(xb, wroot_ref[...], (((1,), (1,)), ((), ())),
                                  preferred_element_type=jnp.float32)


def _tc1(x_pad, w1_rel, w1_root):
    return pl.pallas_call(
        _tc1_body,
        grid=(GRID,),
        in_specs=[
            pl.BlockSpec((BLK, D_IN), lambda i: (i, 0)),
            pl.BlockSpec((D_H, D_IN), lambda i: (0, 0)),
            pl.BlockSpec((D_H, D_IN), lambda i: (0, 0)),
        ],
        out_specs=[
            pl.BlockSpec((BLK, D_H), lambda i: (i, 0)),
            pl.BlockSpec((BLK, D_H), lambda i: (i, 0)),
        ],
        out_shape=[
            jax.ShapeDtypeStruct((N_PAD, D_H), jnp.float32),
            jax.ShapeDtypeStruct((N_PAD, D_H), jnp.float32),
        ],
    )(x_pad, w1_rel, w1_root)


def _tc2_body(acc_ref, r1_ref, b1_ref, ms_ref, h_ref):
    a0 = acc_ref[0]
    a1 = acc_ref[1]
    seg = a0[:, :D_H] + a1[:, :D_H]
    deg = a0[:, D_H:D_H + 1] + a1[:, D_H:D_H + 1]
    degc = jnp.maximum(deg, 1.0)
    h = seg / degc + b1_ref[...] + r1_ref[...]
    h_ref[...] = jnp.maximum(h, 0.0) * ms_ref[...]


def _tc2(acc1, r1, b1r, ms_pad):
    return pl.pallas_call(
        _tc2_body,
        grid=(GRID,),
        in_specs=[
            pl.BlockSpec((2, BLK, 128), lambda i: (0, i, 0)),
            pl.BlockSpec((BLK, D_H), lambda i: (i, 0)),
            pl.BlockSpec((1, D_H), lambda i: (0, 0)),
            pl.BlockSpec((BLK, D_H), lambda i: (i, 0)),
        ],
        out_specs=pl.BlockSpec((BLK, D_H), lambda i: (i, 0)),
        out_shape=jax.ShapeDtypeStruct((N_PAD, D_H), jnp.float32),
    )(acc1, r1, b1r, ms_pad)


def _tc3_body(acc_ref, h_ref, wrel_ref, b2_ref, wroot_ref, out_ref):
    seg2 = acc_ref[0][:, :D_H] + acc_ref[1][:, :D_H]
    out = lax.dot_general(seg2, wrel_ref[...], (((1,), (1,)), ((), ())),
                          preferred_element_type=jnp.float32)
    out += lax.dot_general(h_ref[...], wroot_ref[...], (((1,), (1,)), ((), ())),
                           preferred_element_type=jnp.float32)
    out_ref[...] = out + b2_ref[...]


def _tc3(acc2, h, w2_rel, b2r, w2_root):
    return pl.pallas_call(
        _tc3_body,
        grid=(GRID,),
        in_specs=[
            pl.BlockSpec((2, BLK, 128), lambda i: (0, i, 0)),
            pl.BlockSpec((BLK, D_H), lambda i: (i, 0)),
            pl.BlockSpec((D_OUT, D_H), lambda i: (0, 0)),
            pl.BlockSpec((1, D_OUT), lambda i: (0, 0)),
            pl.BlockSpec((D_OUT, D_H), lambda i: (0, 0)),
        ],
        out_specs=pl.BlockSpec((BLK, D_OUT), lambda i: (i, 0)),
        out_shape=jax.ShapeDtypeStruct((N_PAD, D_OUT), jnp.float32),
    )(acc2, h, w2_rel, b2r, w2_root)


def _make_sc_seg_sum(with_deg):
    width = D_H
    mesh = plsc.VectorSubcoreMesh(core_axis_name="c", subcore_axis_name="s")
    scratch = [
        pltpu.VMEM((CPW, CHUNK), jnp.int32),
        pltpu.VMEM((CPW, CHUNK), jnp.int32),
        [pltpu.VMEM((CHUNK, width), jnp.float32)] * 4,
        pltpu.VMEM_SHARED((N_PAD, width), jnp.float32),
        [pltpu.SemaphoreType.DMA] * 4,
    ]
    if with_deg:
        scratch += [
            pltpu.VMEM((CHUNK, 16), jnp.float32),
            pltpu.VMEM_SHARED((N_PAD, 16), jnp.float32),
        ]

    @functools.partial(
        pl.kernel,
        mesh=mesh,
        compiler_params=pltpu.CompilerParams(use_tc_tiling_on_sc=False),
        out_type=jax.ShapeDtypeStruct((NC, N_PAD, 128), jnp.float32),
        scratch_types=scratch,
    )
    def seg_sum(table_hbm, src_hbm, dst_hbm, out_hbm,
                src_all, dst_all, rows, acc_sh, gsem, *deg_refs):
        if with_deg:
            ones_v, deg_sh = deg_refs
        c = lax.axis_index("c")
        s = lax.axis_index("s")
        wid = s * NC + c

        def _zero_row(i, carry):
            for jj in range(width // 16):
                rows[0][i, pl.ds(jj * 16, 16)] = jnp.zeros((16,), jnp.float32)
            if with_deg:
                ones_v[i, pl.ds(0, 16)] = jnp.zeros((16,), jnp.float32)
            return carry

        lax.fori_loop(0, CHUNK, _zero_row, 0)
        for k in range(STRIPE // CHUNK):
            pltpu.sync_copy(rows[0], acc_sh.at[pl.ds(s * STRIPE + k * CHUNK, CHUNK)])
            if with_deg:
                pltpu.sync_copy(
                    ones_v, deg_sh.at[pl.ds(s * STRIPE + k * CHUNK, CHUNK)])
        if with_deg:
            def _ones_row(i, carry):
                ones_v[i, pl.ds(0, 16)] = jnp.ones((16,), jnp.float32)
                return carry

            lax.fori_loop(0, CHUNK, _ones_row, 0)

        pltpu.sync_copy(src_hbm.at[pl.ds(wid * CPW, CPW)], src_all)
        pltpu.sync_copy(dst_hbm.at[pl.ds(wid * CPW, CPW)], dst_all)
        plsc.subcore_barrier()

        def _gather(j, b):
            pltpu.async_copy(table_hbm.at[src_all.at[j]], rows[b], gsem[b])

        def _gwait(j, b):
            pltpu.make_async_copy(
                table_hbm.at[src_all.at[j]], rows[b], gsem[b]).wait()

        def _scatter(j, b):
            pltpu.sync_copy(rows[b], acc_sh.at[dst_all.at[j]], add=True)
            if with_deg:
                pltpu.sync_copy(ones_v, deg_sh.at[dst_all.at[j]], add=True)

        def _step(j, b, fire_ahead):
            if fire_ahead:
                _gather(j + 3, (b + 3) % 4)
            _gwait(j, b)
            _scatter(j, b)

        _gather(0, 0)
        _gather(1, 1)
        _gather(2, 2)
        for b in range(4):
            _step(b, b, fire_ahead=True)

        def _quad(q, carry):
            for b in range(4):
                _step(4 * q + b, b, fire_ahead=True)
            return carry

        lax.fori_loop(1, CPW // 4 - 1, _quad, 0)

        for b in range(4):
            _step(CPW - 4 + b, b, fire_ahead=(b < 1))
        plsc.subcore_barrier()

        pltpu.sync_copy(acc_sh.at[pl.ds(s * STRIPE, STRIPE)],
                        out_hbm.at[c, pl.ds(s * STRIPE, STRIPE), pl.ds(0, D_H)])
        if with_deg:
            pltpu.sync_copy(
                deg_sh.at[pl.ds(s * STRIPE, STRIPE)],
                out_hbm.at[c, pl.ds(s * STRIPE, STRIPE), pl.ds(D_H, 16)])

    return seg_sum


_sc_seg1 = _make_sc_seg_sum(with_deg=True)
_sc_seg2 = _make_sc_seg_sum(with_deg=False)


def kernel(x, edge_index, W1_rel, b1_rel, W1_root, W2_rel, b2_rel, W2_root):
    x_pad = jnp.pad(x, ((0, N_PAD - N), (0, 0)))
    pad_idx = N + jnp.arange(E_PAD - E, dtype=jnp.int32) % (N_PAD - N)
    src = jnp.concatenate([edge_index[0], pad_idx]).reshape(-1, CHUNK)
    dst = jnp.concatenate([edge_index[1], pad_idx]).reshape(-1, CHUNK)

    ms = 2.0 * jax.random.bernoulli(
        jax.random.key(42), 0.5, (N, D_H)).astype(jnp.float32)
    ms_pad = jnp.pad(ms, ((0, N_PAD - N), (0, 0)))

    y1, r1 = _tc1(x_pad, W1_rel, W1_root)
    acc1 = _sc_seg1(y1, src, dst)
    h = _tc2(acc1, r1, b1_rel.reshape(1, D_H), ms_pad)
    acc2 = _sc_seg2(h, src, dst)
    out = _tc3(acc2, h, W2_rel, b2_rel.reshape(1, D_OUT), W2_root)
    return out[:N]


# speedup vs baseline: 20.3937x; 1.0388x over previous
import functools

import jax
import jax.numpy as jnp
from jax import lax
from jax.experimental import pallas as pl
from jax.experimental.pallas import tpu as pltpu
from jax.experimental.pallas import tpu_sc as plsc

N = 10000
E = 320000
D_IN = 128
D_H = 64
D_OUT = 128

N_PAD = 10240
BLK = 2048
GRID = N_PAD // BLK
NC, NS = 2, 16
NW = NC * NS
CHUNK = 128
CPW = 80
E_PAD = NW * CPW * CHUNK
STRIPE = N_PAD // NS
def _tc1_body(x_ref, wrel_ref, wroot_ref, y1_ref, r1_ref):
    xb = x_ref[...]
    y1_ref[...] = lax.dot_general(xb, wrel_ref[...], (((1,), (1,)), ((), ())),
                                  preferred_element_type=jnp.float32)
    r1_ref[...] = lax.dot_general(xb, wroot_ref[...], (((1,), (1,)), ((), ())),
                                  preferred_element_type=jnp.float32)


def _tc1(x, w1_rel, w1_root):
    return pl.pallas_call(
        _tc1_body,
        grid=(GRID,),
        in_specs=[
            pl.BlockSpec((N // GRID, D_IN), lambda i: (i, 0)),
            pl.BlockSpec((D_H, D_IN), lambda i: (0, 0)),
            pl.BlockSpec((D_H, D_IN), lambda i: (0, 0)),
        ],
        out_specs=[
            pl.BlockSpec((N // GRID, D_H), lambda i: (i, 0)),
            pl.BlockSpec((N // GRID, D_H), lambda i: (i, 0)),
        ],
        out_shape=[
            jax.ShapeDtypeStruct((N_PAD, D_H), jnp.float32),
            jax.ShapeDtypeStruct((N_PAD, D_H), jnp.float32),
        ],
    )(x, w1_rel, w1_root)


def _tc2_body(acc_ref, r1_ref, b1_ref, ms_ref, h_ref):
    a0 = acc_ref[0]
    a1 = acc_ref[1]
    seg = a0[:, :D_H] + a1[:, :D_H]
    deg = a0[:, D_H:D_H + 1] + a1[:, D_H:D_H + 1]
    degc = jnp.maximum(deg, 1.0)
    h = seg / degc + b1_ref[...] + r1_ref[...]
    h_ref[...] = jnp.maximum(h, 0.0) * ms_ref[...]


def _tc2(acc1, r1, b1r, ms_pad):
    return pl.pallas_call(
        _tc2_body,
        grid=(GRID,),
        in_specs=[
            pl.BlockSpec((2, BLK, 128), lambda i: (0, i, 0)),
            pl.BlockSpec((BLK, D_H), lambda i: (i, 0)),
            pl.BlockSpec((1, D_H), lambda i: (0, 0)),
            pl.BlockSpec((BLK, D_H), lambda i: (i, 0)),
        ],
        out_specs=pl.BlockSpec((BLK, D_H), lambda i: (i, 0)),
        out_shape=jax.ShapeDtypeStruct((N_PAD, D_H), jnp.float32),
    )(acc1, r1, b1r, ms_pad)


def _tc3_body(acc_ref, h_ref, wrel_ref, b2_ref, wroot_ref, out_ref):
    seg2 = acc_ref[0][:, :D_H] + acc_ref[1][:, :D_H]
    out = lax.dot_general(seg2, wrel_ref[...], (((1,), (1,)), ((), ())),
                          preferred_element_type=jnp.float32)
    out += lax.dot_general(h_ref[...], wroot_ref[...], (((1,), (1,)), ((), ())),
                           preferred_element_type=jnp.float32)
    out_ref[...] = out + b2_ref[...]


def _tc3(acc2, h, w2_rel, b2r, w2_root):
    return pl.pallas_call(
        _tc3_body,
        grid=(GRID,),
        in_specs=[
            pl.BlockSpec((2, N // GRID, 128), lambda i: (0, i, 0)),
            pl.BlockSpec((N // GRID, D_H), lambda i: (i, 0)),
            pl.BlockSpec((D_OUT, D_H), lambda i: (0, 0)),
            pl.BlockSpec((1, D_OUT), lambda i: (0, 0)),
            pl.BlockSpec((D_OUT, D_H), lambda i: (0, 0)),
        ],
        out_specs=pl.BlockSpec((N // GRID, D_OUT), lambda i: (i, 0)),
        out_shape=jax.ShapeDtypeStruct((N, D_OUT), jnp.float32),
    )(acc2, h, w2_rel, b2r, w2_root)


def _make_sc_seg_sum(with_deg):
    width = D_H
    mesh = plsc.VectorSubcoreMesh(core_axis_name="c", subcore_axis_name="s")
    scratch = [
        pltpu.VMEM((CPW, CHUNK), jnp.int32),
        pltpu.VMEM((CPW, CHUNK), jnp.int32),
        [pltpu.VMEM((CHUNK, width), jnp.float32)] * 4,
        pltpu.VMEM_SHARED((N_PAD, width), jnp.float32),
        [pltpu.SemaphoreType.DMA] * 4,
    ]
    if with_deg:
        scratch += [
            pltpu.VMEM((CHUNK, 16), jnp.float32),
            pltpu.VMEM_SHARED((N_PAD, 16), jnp.float32),
        ]

    @functools.partial(
        pl.kernel,
        mesh=mesh,
        compiler_params=pltpu.CompilerParams(use_tc_tiling_on_sc=False),
        out_type=jax.ShapeDtypeStruct((NC, N_PAD, 128), jnp.float32),
        scratch_types=scratch,
    )
    def seg_sum(table_hbm, src_hbm, dst_hbm, out_hbm,
                src_all, dst_all, rows, acc_sh, gsem, *deg_refs):
        if with_deg:
            ones_v, deg_sh = deg_refs
        c = lax.axis_index("c")
        s = lax.axis_index("s")
        wid = s * NC + c

        def _zero_row(i, carry):
            for jj in range(width // 16):
                rows[0][i, pl.ds(jj * 16, 16)] = jnp.zeros((16,), jnp.float32)
            if with_deg:
                ones_v[i, pl.ds(0, 16)] = jnp.zeros((16,), jnp.float32)
            return carry

        lax.fori_loop(0, CHUNK, _zero_row, 0)
        for k in range(STRIPE // CHUNK):
            pltpu.sync_copy(rows[0], acc_sh.at[pl.ds(s * STRIPE + k * CHUNK, CHUNK)])
            if with_deg:
                pltpu.sync_copy(
                    ones_v, deg_sh.at[pl.ds(s * STRIPE + k * CHUNK, CHUNK)])
        if with_deg:
            def _ones_row(i, carry):
                ones_v[i, pl.ds(0, 16)] = jnp.ones((16,), jnp.float32)
                return carry

            lax.fori_loop(0, CHUNK, _ones_row, 0)

        pltpu.sync_copy(src_hbm.at[pl.ds(wid * CPW, CPW)], src_all)
        pltpu.sync_copy(dst_hbm.at[pl.ds(wid * CPW, CPW)], dst_all)
        plsc.subcore_barrier()

        def _gather(j, b):
            pltpu.async_copy(table_hbm.at[src_all.at[j]], rows[b], gsem[b])

        def _gwait(j, b):
            pltpu.make_async_copy(
                table_hbm.at[src_all.at[j]], rows[b], gsem[b]).wait()

        def _scatter(j, b):
            pltpu.sync_copy(rows[b], acc_sh.at[dst_all.at[j]], add=True)
            if with_deg:
                pltpu.sync_copy(ones_v, deg_sh.at[dst_all.at[j]], add=True)

        def _step(j, b, fire_ahead):
            if fire_ahead:
                _gather(j + 3, (b + 3) % 4)
            _gwait(j, b)
            _scatter(j, b)

        _gather(0, 0)
        _gather(1, 1)
        _gather(2, 2)
        for b in range(4):
            _step(b, b, fire_ahead=True)

        def _quad(q, carry):
            for b in range(4):
                _step(4 * q + b, b, fire_ahead=True)
            return carry

        lax.fori_loop(1, CPW // 4 - 1, _quad, 0)

        for b in range(4):
            _step(CPW - 4 + b, b, fire_ahead=(b < 1))
        plsc.subcore_barrier()

        pltpu.sync_copy(acc_sh.at[pl.ds(s * STRIPE, STRIPE)],
                        out_hbm.at[c, pl.ds(s * STRIPE, STRIPE), pl.ds(0, D_H)])
        if with_deg:
            pltpu.sync_copy(
                deg_sh.at[pl.ds(s * STRIPE, STRIPE)],
                out_hbm.at[c, pl.ds(s * STRIPE, STRIPE), pl.ds(D_H, 16)])

    return seg_sum


_sc_seg1 = _make_sc_seg_sum(with_deg=True)
_sc_seg2 = _make_sc_seg_sum(with_deg=False)


def kernel(x, edge_index, W1_rel, b1_rel, W1_root, W2_rel, b2_rel, W2_root):
    pad_idx = N + jnp.arange(E_PAD - E, dtype=jnp.int32) % (N_PAD - N)
    src = jnp.concatenate([edge_index[0], pad_idx]).reshape(-1, CHUNK)
    dst = jnp.concatenate([edge_index[1], pad_idx]).reshape(-1, CHUNK)

    ms = 2.0 * jax.random.bernoulli(
        jax.random.key(42), 0.5, (N, D_H)).astype(jnp.float32)
    ms_pad = jnp.pad(ms, ((0, N_PAD - N), (0, 0)))

    y1, r1 = _tc1(x, W1_rel, W1_root)
    acc1 = _sc_seg1(y1, src, dst)
    h = _tc2(acc1, r1, b1_rel.reshape(1, D_H), ms_pad)
    acc2 = _sc_seg2(h, src, dst)
    return _tc3(acc2, h, W2_rel, b2_rel.reshape(1, D_OUT), W2_root)
